# baseline (speedup 1.0000x reference)
import functools

import jax
import jax.numpy as jnp
from jax import lax
from jax.experimental import pallas as pl
from jax.experimental.pallas import tpu as pltpu
from jax.experimental.pallas import tpu_sc as plsc

D = 128
L = 16
NC, NS = 2, 16
NW = NC * NS
CH = 80


@functools.partial(jax.jit, static_argnames=("E",))
def _dot_scores(table, src, dst, E):
    EPW = E // NW
    NCH = EPW // CH

    mesh = plsc.VectorSubcoreMesh(
        core_axis_name="c", subcore_axis_name="s", num_cores=NC, num_subcores=NS)

    @functools.partial(
        pl.kernel,
        out_type=jax.ShapeDtypeStruct((E,), jnp.float32),
        mesh=mesh,
        compiler_params=pltpu.CompilerParams(needs_layout_passes=False),
        scratch_types=[
            pltpu.VMEM((CH,), jnp.int32),
            pltpu.VMEM((CH,), jnp.int32),
            pltpu.VMEM((CH, D), jnp.float32),
            pltpu.VMEM((CH, D), jnp.float32),
            pltpu.VMEM((CH,), jnp.float32),
            pltpu.SemaphoreType.DMA,
            pltpu.SemaphoreType.DMA,
        ],
    )
    def k(table_hbm, src_hbm, dst_hbm, out_hbm,
          sidx, didx, srows, drows, outv, sem1, sem2):
        wid = lax.axis_index("s") * NC + lax.axis_index("c")
        base = wid * EPW

        def chunk_body(c, carry):
            off = base + c * CH
            pltpu.sync_copy(src_hbm.at[pl.ds(off, CH)], sidx)
            pltpu.sync_copy(dst_hbm.at[pl.ds(off, CH)], didx)
            g1 = pltpu.async_copy(table_hbm.at[sidx], srows, sem1)
            g2 = pltpu.async_copy(table_hbm.at[didx], drows, sem2)
            g1.wait()
            g2.wait()

            def group_body(g, carry2):
                rows = jnp.full((L,), g * L, jnp.int32) + lax.iota(jnp.int32, L)
                accs = [jnp.zeros((L,), jnp.float32) for _ in range(4)]
                for d in range(D):
                    col = jnp.full((L,), d, jnp.int32)
                    s = plsc.load_gather(srows, [rows, col])
                    t = plsc.load_gather(drows, [rows, col])
                    accs[d % 4] = accs[d % 4] + s * t
                outv[pl.ds(g * L, L)] = (accs[0] + accs[1]) + (accs[2] + accs[3])
                return carry2

            lax.fori_loop(0, CH // L, group_body, 0, unroll=False)
            pltpu.sync_copy(outv, out_hbm.at[pl.ds(off, CH)])
            return carry

        lax.fori_loop(0, NCH, chunk_body, 0, unroll=False)

    return k(table, src, dst)


def kernel(node_embeddings, edge_index):
    idx = edge_index.astype(jnp.int32)
    E = idx.shape[1]
    scores = _dot_scores(node_embeddings, idx[0], idx[1], E)
    return scores.reshape(E, 1)


# speedup vs baseline: 1.1900x; 1.1900x over previous
import functools

import jax
import jax.numpy as jnp
from jax import lax
from jax.experimental import pallas as pl
from jax.experimental.pallas import tpu as pltpu
from jax.experimental.pallas import tpu_sc as plsc

D = 128
L = 16
NC, NS = 2, 16
NW = NC * NS
CH = 80
NB = 2


@functools.partial(jax.jit, static_argnames=("E",))
def _dot_scores(table, src, dst, E):
    EPW = E // NW
    NCH = EPW // CH

    mesh = plsc.VectorSubcoreMesh(
        core_axis_name="c", subcore_axis_name="s", num_cores=NC, num_subcores=NS)

    row_bufs = [pltpu.VMEM((CH, D), jnp.float32) for _ in range(2 * NB)]
    sems = [pltpu.SemaphoreType.DMA for _ in range(NB)]

    @functools.partial(
        pl.kernel,
        out_type=jax.ShapeDtypeStruct((E,), jnp.float32),
        mesh=mesh,
        compiler_params=pltpu.CompilerParams(needs_layout_passes=False),
        scratch_types=[
            pltpu.VMEM((EPW,), jnp.int32),
            pltpu.VMEM((EPW,), jnp.int32),
            pltpu.VMEM((EPW,), jnp.float32),
        ] + row_bufs + sems,
    )
    def k(table_hbm, src_hbm, dst_hbm, out_hbm, sidx, didx, outv, *bufs_and_sems):
        bufs = [(bufs_and_sems[2 * b], bufs_and_sems[2 * b + 1])
                for b in range(NB)]
        sem = bufs_and_sems[2 * NB:]
        wid = lax.axis_index("s") * NC + lax.axis_index("c")
        base = wid * EPW

        pltpu.sync_copy(src_hbm.at[pl.ds(base, EPW)], sidx)
        pltpu.sync_copy(dst_hbm.at[pl.ds(base, EPW)], didx)

        def fire(c, b):
            sb, db = bufs[b]
            pltpu.async_copy(table_hbm.at[sidx.at[pl.ds(c * CH, CH)]], sb,
                             sem[b])
            pltpu.async_copy(table_hbm.at[didx.at[pl.ds(c * CH, CH)]], db,
                             sem[b])

        def drain(c, b):
            sb, db = bufs[b]
            pltpu.make_async_copy(
                table_hbm.at[sidx.at[pl.ds(c * CH, CH)]], sb, sem[b]).wait()
            pltpu.make_async_copy(
                table_hbm.at[didx.at[pl.ds(c * CH, CH)]], db, sem[b]).wait()

        for b in range(NB):
            fire(b, b)

        def chunk_body(c, carry):
            for b in range(NB):
                @pl.when(lax.rem(c, NB) == b)
                def _(b=b):
                    drain(c, b)
                    sb, db = bufs[b]

                    def group_body(g, carry2):
                        rows = (jnp.full((L,), g * L, jnp.int32)
                                + lax.iota(jnp.int32, L))
                        accs = [jnp.zeros((L,), jnp.float32) for _ in range(4)]
                        for d in range(D):
                            col = jnp.full((L,), d, jnp.int32)
                            s = plsc.load_gather(sb, [rows, col])
                            t = plsc.load_gather(db, [rows, col])
                            accs[d % 4] = accs[d % 4] + s * t
                        outv[pl.ds(c * CH + g * L, L)] = (
                            (accs[0] + accs[1]) + (accs[2] + accs[3]))
                        return carry2

                    lax.fori_loop(0, CH // L, group_body, 0, unroll=False)

                    @pl.when(c + NB < NCH)
                    def _():
                        fire(c + NB, b)
            return carry

        lax.fori_loop(0, NCH, chunk_body, 0, unroll=False)
        pltpu.sync_copy(outv, out_hbm.at[pl.ds(base, EPW)])

    return k(table, src, dst)


def kernel(node_embeddings, edge_index):
    idx = edge_index.astype(jnp.int32)
    E = idx.shape[1]
    scores = _dot_scores(node_embeddings, idx[0], idx[1], E)
    return scores.reshape(E, 1)


# speedup vs baseline: 6.5776x; 5.5273x over previous
import functools

import jax
import jax.numpy as jnp
from jax import lax
from jax.experimental import pallas as pl
from jax.experimental.pallas import tpu as pltpu
from jax.experimental.pallas import tpu_sc as plsc

D = 128
L = 16
NC, NS = 2, 16
NW = NC * NS
CH = 80
NB = 2


@functools.partial(jax.jit, static_argnames=("E",))
def _dot_scores(table, src, dst, E):
    EPW = E // NW
    NCH = EPW // CH

    mesh = plsc.VectorSubcoreMesh(
        core_axis_name="c", subcore_axis_name="s", num_cores=NC, num_subcores=NS)

    row_bufs = [pltpu.VMEM((CH, D), jnp.float32) for _ in range(2 * NB)]
    sems = [pltpu.SemaphoreType.DMA for _ in range(NB)]

    @functools.partial(
        pl.kernel,
        out_type=jax.ShapeDtypeStruct((E,), jnp.float32),
        mesh=mesh,
        compiler_params=pltpu.CompilerParams(needs_layout_passes=False),
        scratch_types=[
            pltpu.VMEM((EPW,), jnp.int32),
            pltpu.VMEM((EPW,), jnp.int32),
            pltpu.VMEM((EPW + L,), jnp.float32),
        ] + row_bufs + sems,
    )
    def k(table_hbm, src_hbm, dst_hbm, out_hbm, sidx, didx, outv, *bufs_and_sems):
        bufs = [(bufs_and_sems[2 * b], bufs_and_sems[2 * b + 1])
                for b in range(NB)]
        sem = bufs_and_sems[2 * NB:]
        wid = lax.axis_index("s") * NC + lax.axis_index("c")
        base = wid * EPW

        pltpu.sync_copy(src_hbm.at[pl.ds(base, EPW)], sidx)
        pltpu.sync_copy(dst_hbm.at[pl.ds(base, EPW)], didx)

        def fire(c, b):
            sb, db = bufs[b]
            pltpu.async_copy(table_hbm.at[sidx.at[pl.ds(c * CH, CH)]], sb,
                             sem[b])
            pltpu.async_copy(table_hbm.at[didx.at[pl.ds(c * CH, CH)]], db,
                             sem[b])

        def drain(c, b):
            sb, db = bufs[b]
            pltpu.make_async_copy(
                table_hbm.at[sidx.at[pl.ds(c * CH, CH)]], sb, sem[b]).wait()
            pltpu.make_async_copy(
                table_hbm.at[didx.at[pl.ds(c * CH, CH)]], db, sem[b]).wait()

        for b in range(NB):
            fire(b, b)

        def chunk_body(c, carry):
            for b in range(NB):
                @pl.when(lax.rem(c, NB) == b)
                def _(b=b):
                    drain(c, b)
                    sb, db = bufs[b]

                    last_lane = lax.iota(jnp.int32, L) == (L - 1)

                    def edge_body(e, carry2):
                        accs = [jnp.zeros((L,), jnp.float32) for _ in range(2)]
                        for j in range(D // L):
                            s = sb[e, pl.ds(j * L, L)]
                            t = db[e, pl.ds(j * L, L)]
                            accs[j % 2] = accs[j % 2] + s * t
                        red = plsc.cumsum(accs[0] + accs[1])
                        plsc.store_compressed(
                            outv.at[pl.ds(c * CH + e, L)], red, mask=last_lane)
                        return carry2

                    lax.fori_loop(0, CH, edge_body, 0, unroll=4)

                    @pl.when(c + NB < NCH)
                    def _():
                        fire(c + NB, b)
            return carry

        lax.fori_loop(0, NCH, chunk_body, 0, unroll=False)
        pltpu.sync_copy(outv.at[pl.ds(0, EPW)], out_hbm.at[pl.ds(base, EPW)])

    return k(table, src, dst)


def kernel(node_embeddings, edge_index):
    idx = edge_index.astype(jnp.int32)
    E = idx.shape[1]
    scores = _dot_scores(node_embeddings, idx[0], idx[1], E)
    return scores.reshape(E, 1)
